# speedup vs baseline: 7.7221x; 7.7221x over previous
import functools
import jax
import jax.numpy as jnp
from jax.experimental import pallas as pl
from jax.experimental.pallas import tpu as pltpu

_B = 16
_D = 64


def _t16(row):
    eye = (jax.lax.broadcasted_iota(jnp.int32, (_B, _B), 0)
           == jax.lax.broadcasted_iota(jnp.int32, (_B, _B), 1)).astype(row.dtype)
    return jnp.sum(eye * row, axis=1, keepdims=True)


def _fused_body(seg_ref, x_ref, u_ref, wu_ref, bu_ref, wk_ref, we_ref,
                out_ref, m_scr, s_scr, acc_scr):
    i = pl.program_id(0)
    nb = pl.num_programs(0)
    C = x_ref.shape[0]

    @pl.when(i == 0)
    def _init():
        m_scr[...] = jnp.full(m_scr.shape, -jnp.inf, m_scr.dtype)
        s_scr[...] = jnp.zeros(s_scr.shape, s_scr.dtype)
        acc_scr[...] = jnp.zeros(acc_scr.shape, acc_scr.dtype)

    x = x_ref[...]
    seg = seg_ref[...]
    u = jnp.dot(u_ref[...], wu_ref[...],
                preferred_element_type=jnp.float32) + bu_ref[...]
    key = jnp.dot(x, wk_ref[...], preferred_element_type=jnp.float32)
    seg_eq = seg == jax.lax.broadcasted_iota(jnp.int32, (C, _B), 1)
    onehot = seg_eq.astype(jnp.float32)
    qry = jnp.dot(onehot, u, preferred_element_type=jnp.float32)
    z = jax.nn.sigmoid(qry + key)
    e = jnp.sum(z * we_ref[...], axis=1, keepdims=True)

    em = jnp.where(seg_eq, jnp.broadcast_to(e, (C, _B)), -jnp.inf)
    cm = jnp.max(em, axis=0, keepdims=True)
    m_old = m_scr[...]
    m_new = jnp.maximum(m_old, cm)
    scale = jnp.where(m_old == -jnp.inf, 0.0, jnp.exp(m_old - m_new))
    m_safe = jnp.where(m_new == -jnp.inf, 0.0, m_new)
    row_m = jnp.sum(onehot * m_safe, axis=1, keepdims=True)
    p = jnp.exp(e - row_m)
    wh = onehot * p
    s_new = s_scr[...] * scale + jnp.sum(wh, axis=0, keepdims=True)
    acc_chunk = jax.lax.dot_general(wh, x, (((0,), (0,)), ((), ())),
                                    preferred_element_type=jnp.float32)
    acc_new = acc_scr[...] * _t16(scale) + acc_chunk

    m_scr[...] = m_new
    s_scr[...] = s_new
    acc_scr[...] = acc_new

    @pl.when(i == nb - 1)
    def _fin():
        s_col = _t16(s_new)
        out_ref[...] = jnp.where(s_col > 0, acc_new / s_col, 0.0)


def kernel(feat_i, feat_u, segment_ids, W_user, b_user, W_key, W_e):
    N, D = feat_i.shape
    C = 4096
    grid = (N // C,)
    seg2d = segment_ids.reshape(N, 1)
    bu = b_user.reshape(1, D)
    we = W_e.reshape(1, D)

    return pl.pallas_call(
        _fused_body,
        grid=grid,
        in_specs=[
            pl.BlockSpec((C, 1), lambda i: (i, 0)),
            pl.BlockSpec((C, D), lambda i: (i, 0)),
            pl.BlockSpec((_B, D), lambda i: (0, 0)),
            pl.BlockSpec((D, D), lambda i: (0, 0)),
            pl.BlockSpec((1, D), lambda i: (0, 0)),
            pl.BlockSpec((D, D), lambda i: (0, 0)),
            pl.BlockSpec((1, D), lambda i: (0, 0)),
        ],
        out_specs=pl.BlockSpec((_B, D), lambda i: (0, 0)),
        out_shape=jax.ShapeDtypeStruct((_B, D), jnp.float32),
        scratch_shapes=[
            pltpu.VMEM((1, _B), jnp.float32),
            pltpu.VMEM((1, _B), jnp.float32),
            pltpu.VMEM((_B, D), jnp.float32),
        ],
        compiler_params=pltpu.CompilerParams(
            dimension_semantics=("arbitrary",),
        ),
    )(seg2d, feat_i, feat_u, W_user, bu, W_key, we)


# speedup vs baseline: 10.9842x; 1.4224x over previous
import jax
import jax.numpy as jnp
from jax.experimental import pallas as pl
from jax.experimental.pallas import tpu as pltpu

_B = 16
_D = 64


def _fused_body(seg_ref, x_ref, u_ref, wu_ref, bu_ref, wk_ref, we_ref,
                out_ref, u_scr, m_scr, s_scr, acc_scr):
    i = pl.program_id(0)
    nb = pl.num_programs(0)
    C = x_ref.shape[0]

    @pl.when(i == 0)
    def _init():
        u_scr[...] = jnp.dot(u_ref[...], wu_ref[...],
                             preferred_element_type=jnp.float32) + bu_ref[...]
        m_scr[...] = jnp.full(m_scr.shape, -jnp.inf, m_scr.dtype)
        s_scr[...] = jnp.zeros(s_scr.shape, s_scr.dtype)
        acc_scr[...] = jnp.zeros(acc_scr.shape, acc_scr.dtype)

    x = x_ref[...]
    seg_row = seg_ref[0]
    onehot_b = seg_row == jax.lax.broadcasted_iota(jnp.int32, (_B, C), 0)
    onehot = onehot_b.astype(jnp.float32)

    key = jnp.dot(x, wk_ref[...], preferred_element_type=jnp.float32)
    qry = jax.lax.dot_general(onehot, u_scr[...], (((0,), (0,)), ((), ())),
                              preferred_element_type=jnp.float32)
    z = jax.nn.sigmoid(key + qry)
    e_row = jax.lax.dot_general(we_ref[...], z, (((1,), (1,)), ((), ())),
                                preferred_element_type=jnp.float32)

    em = jnp.where(onehot_b, jnp.broadcast_to(e_row, (_B, C)), -jnp.inf)
    cm = jnp.max(em, axis=1, keepdims=True)
    m_old = m_scr[...]
    m_new = jnp.maximum(m_old, cm)
    scale = jnp.where(m_old == -jnp.inf, 0.0, jnp.exp(m_old - m_new))
    m_safe = jnp.where(m_new == -jnp.inf, 0.0, m_new)
    row_m = jax.lax.dot_general(m_safe, onehot, (((0,), (0,)), ((), ())),
                                preferred_element_type=jnp.float32)
    p_row = jnp.exp(e_row - row_m)
    whT = onehot * p_row
    s_new = s_scr[...] * scale + jnp.sum(whT, axis=1, keepdims=True)
    acc_chunk = jnp.dot(whT, x, preferred_element_type=jnp.float32)
    acc_new = acc_scr[...] * scale + acc_chunk

    m_scr[...] = m_new
    s_scr[...] = s_new
    acc_scr[...] = acc_new

    @pl.when(i == nb - 1)
    def _fin():
        out_ref[...] = jnp.where(s_new > 0, acc_new / s_new, 0.0)


def kernel(feat_i, feat_u, segment_ids, W_user, b_user, W_key, W_e):
    N, D = feat_i.shape
    C = 4096
    nb = N // C
    seg3 = segment_ids.reshape(nb, 1, C)
    bu = b_user.reshape(1, D)
    we = W_e.reshape(1, D)

    return pl.pallas_call(
        _fused_body,
        grid=(nb,),
        in_specs=[
            pl.BlockSpec((1, 1, C), lambda i: (i, 0, 0)),
            pl.BlockSpec((C, D), lambda i: (i, 0)),
            pl.BlockSpec((_B, D), lambda i: (0, 0)),
            pl.BlockSpec((D, D), lambda i: (0, 0)),
            pl.BlockSpec((1, D), lambda i: (0, 0)),
            pl.BlockSpec((D, D), lambda i: (0, 0)),
            pl.BlockSpec((1, D), lambda i: (0, 0)),
        ],
        out_specs=pl.BlockSpec((_B, D), lambda i: (0, 0)),
        out_shape=jax.ShapeDtypeStruct((_B, D), jnp.float32),
        scratch_shapes=[
            pltpu.VMEM((_B, _D), jnp.float32),
            pltpu.VMEM((_B, 1), jnp.float32),
            pltpu.VMEM((_B, 1), jnp.float32),
            pltpu.VMEM((_B, _D), jnp.float32),
        ],
        compiler_params=pltpu.CompilerParams(
            dimension_semantics=("arbitrary",),
        ),
    )(seg3, feat_i, feat_u, W_user, bu, W_key, we)


# speedup vs baseline: 11.1674x; 1.0167x over previous
import functools
import jax
import jax.numpy as jnp
from jax.experimental import pallas as pl
from jax.experimental.pallas import tpu as pltpu

_B = 16
_D = 64


def _eye(dtype=jnp.float32):
    return (jax.lax.broadcasted_iota(jnp.int32, (_B, _B), 0)
            == jax.lax.broadcasted_iota(jnp.int32, (_B, _B), 1)).astype(dtype)


def _row_of(col):
    return jnp.sum(_eye() * col, axis=0, keepdims=True)


def _col_of(row):
    return jnp.sum(_eye() * row, axis=1, keepdims=True)


def _fused_body(nb, seg_ref, x_ref, u_ref, wu_ref, bu_ref, wk_ref, we_ref,
                out_ref, u_scr, m_parts, s_parts, acc_parts):
    i = pl.program_id(0)
    C = x_ref.shape[0]

    @pl.when(i == 0)
    def _init():
        u_scr[...] = jnp.dot(u_ref[...], wu_ref[...],
                             preferred_element_type=jnp.float32) + bu_ref[...]

    x = x_ref[...]
    seg_row = seg_ref[0]
    onehot_b = seg_row == jax.lax.broadcasted_iota(jnp.int32, (_B, C), 0)
    onehot = onehot_b.astype(jnp.float32)

    key = jnp.dot(x, wk_ref[...], preferred_element_type=jnp.float32)
    qry = jax.lax.dot_general(onehot, u_scr[...], (((0,), (0,)), ((), ())),
                              preferred_element_type=jnp.float32)
    th = jnp.tanh((key + qry) * 0.5)
    we_sum = jnp.sum(we_ref[...], axis=1, keepdims=True)
    e_row = 0.5 * (jax.lax.dot_general(we_ref[...], th, (((1,), (1,)), ((), ())),
                                       preferred_element_type=jnp.float32)
                   + we_sum)

    em = jnp.where(onehot_b, jnp.broadcast_to(e_row, (_B, C)), -jnp.inf)
    cm_col = jnp.max(em, axis=1, keepdims=True)
    cm_safe = jnp.where(cm_col == -jnp.inf, 0.0, cm_col)
    cm_row = _row_of(cm_safe)
    row_cm = jnp.dot(cm_row, onehot, preferred_element_type=jnp.float32)
    p_row = jnp.exp(e_row - row_cm)
    whT = onehot * p_row
    s_col = jnp.sum(whT, axis=1, keepdims=True)
    acc_loc = jnp.dot(whT, x, preferred_element_type=jnp.float32)

    m_parts[pl.ds(i, 1), :] = cm_row
    s_parts[pl.ds(i, 1), :] = _row_of(s_col)
    acc_parts[pl.ds(i * _B, _B), :] = acc_loc

    @pl.when(i == nb - 1)
    def _fin():
        mp = m_parts[...]
        sp = s_parts[...]
        m_row = jnp.max(jnp.where(sp > 0, mp, -jnp.inf), axis=0, keepdims=True)
        m_safe = jnp.where(m_row == -jnp.inf, 0.0, m_row)
        scp = jnp.exp(mp - m_safe)
        s_tot = jnp.sum(sp * scp, axis=0, keepdims=True)
        acc_tot = jnp.zeros((_B, _D), jnp.float32)
        for j in range(nb):
            sc_col = _col_of(scp[j:j + 1, :])
            acc_tot = acc_tot + sc_col * acc_parts[j * _B:(j + 1) * _B, :]
        s_ct = _col_of(s_tot)
        out_ref[...] = jnp.where(s_ct > 0, acc_tot / s_ct, 0.0)


def kernel(feat_i, feat_u, segment_ids, W_user, b_user, W_key, W_e):
    N, D = feat_i.shape
    C = 4096
    nb = N // C
    seg3 = segment_ids.reshape(nb, 1, C)
    bu = b_user.reshape(1, D)
    we = W_e.reshape(1, D)

    return pl.pallas_call(
        functools.partial(_fused_body, nb),
        grid=(nb,),
        in_specs=[
            pl.BlockSpec((1, 1, C), lambda i: (i, 0, 0)),
            pl.BlockSpec((C, D), lambda i: (i, 0)),
            pl.BlockSpec((_B, D), lambda i: (0, 0)),
            pl.BlockSpec((D, D), lambda i: (0, 0)),
            pl.BlockSpec((1, D), lambda i: (0, 0)),
            pl.BlockSpec((D, D), lambda i: (0, 0)),
            pl.BlockSpec((1, D), lambda i: (0, 0)),
        ],
        out_specs=pl.BlockSpec((_B, D), lambda i: (0, 0)),
        out_shape=jax.ShapeDtypeStruct((_B, D), jnp.float32),
        scratch_shapes=[
            pltpu.VMEM((_B, _D), jnp.float32),
            pltpu.VMEM((nb, _B), jnp.float32),
            pltpu.VMEM((nb, _B), jnp.float32),
            pltpu.VMEM((nb * _B, _D), jnp.float32),
        ],
        compiler_params=pltpu.CompilerParams(
            dimension_semantics=("arbitrary",),
        ),
    )(seg3, feat_i, feat_u, W_user, bu, W_key, we)


# speedup vs baseline: 11.9697x; 1.0718x over previous
import functools
import jax
import jax.numpy as jnp
from jax.experimental import pallas as pl
from jax.experimental.pallas import tpu as pltpu

_B = 16
_D = 64


def _eye(dtype=jnp.float32):
    return (jax.lax.broadcasted_iota(jnp.int32, (_B, _B), 0)
            == jax.lax.broadcasted_iota(jnp.int32, (_B, _B), 1)).astype(dtype)


def _row_of(col):
    return jnp.sum(_eye() * col, axis=0, keepdims=True)


def _col_of(row):
    return jnp.sum(_eye() * row, axis=1, keepdims=True)


def _fused_body(nb, seg_ref, x_ref, u_ref, wu_ref, bu_ref, wk_ref, we_ref,
                out_ref, u_scr, m_parts, s_parts, acc_parts):
    i = pl.program_id(0)
    C = x_ref.shape[0]

    @pl.when(i == 0)
    def _init():
        u_scr[...] = jnp.dot(u_ref[...], wu_ref[...],
                             preferred_element_type=jnp.float32) + bu_ref[...]

    x = x_ref[...]
    seg_row = seg_ref[0]
    onehot_b = seg_row == jax.lax.broadcasted_iota(jnp.int32, (_B, C), 0)
    onehot = onehot_b.astype(jnp.float32)

    key = jnp.dot(x, wk_ref[...], preferred_element_type=jnp.float32)
    qry = jax.lax.dot_general(onehot, u_scr[...], (((0,), (0,)), ((), ())),
                              preferred_element_type=jnp.float32)
    th = jnp.tanh((key + qry) * 0.5)
    we_sum = jnp.sum(we_ref[...], axis=1, keepdims=True)
    e_row = 0.5 * (jax.lax.dot_general(we_ref[...], th, (((1,), (1,)), ((), ())),
                                       preferred_element_type=jnp.float32)
                   + we_sum)

    em = jnp.where(onehot_b, jnp.broadcast_to(e_row, (_B, C)), -jnp.inf)
    cm_col = jnp.max(em, axis=1, keepdims=True)
    cm_safe = jnp.where(cm_col == -jnp.inf, 0.0, cm_col)
    cm_row = _row_of(cm_safe)
    row_cm = jnp.dot(cm_row, onehot, preferred_element_type=jnp.float32)
    p_row = jnp.exp(e_row - row_cm)
    whT = onehot * p_row
    s_col = jnp.sum(whT, axis=1, keepdims=True)
    acc_loc = jnp.dot(whT, x, preferred_element_type=jnp.float32)

    m_parts[pl.ds(i, 1), :] = cm_row
    s_parts[pl.ds(i, 1), :] = _row_of(s_col)
    acc_parts[pl.ds(i * _B, _B), :] = acc_loc

    @pl.when(i == nb - 1)
    def _fin():
        mp = m_parts[...]
        sp = s_parts[...]
        m_row = jnp.max(jnp.where(sp > 0, mp, -jnp.inf), axis=0, keepdims=True)
        m_safe = jnp.where(m_row == -jnp.inf, 0.0, m_row)
        scp = jnp.exp(mp - m_safe)
        s_tot = jnp.sum(sp * scp, axis=0, keepdims=True)
        acc_tot = jnp.zeros((_B, _D), jnp.float32)
        for j in range(nb):
            sc_col = _col_of(scp[j:j + 1, :])
            acc_tot = acc_tot + sc_col * acc_parts[j * _B:(j + 1) * _B, :]
        s_ct = _col_of(s_tot)
        out_ref[...] = jnp.where(s_ct > 0, acc_tot / s_ct, 0.0)


def kernel(feat_i, feat_u, segment_ids, W_user, b_user, W_key, W_e):
    N, D = feat_i.shape
    C = 8192
    nb = N // C
    seg3 = segment_ids.reshape(nb, 1, C)
    bu = b_user.reshape(1, D)
    we = W_e.reshape(1, D)

    return pl.pallas_call(
        functools.partial(_fused_body, nb),
        grid=(nb,),
        in_specs=[
            pl.BlockSpec((1, 1, C), lambda i: (i, 0, 0)),
            pl.BlockSpec((C, D), lambda i: (i, 0)),
            pl.BlockSpec((_B, D), lambda i: (0, 0)),
            pl.BlockSpec((D, D), lambda i: (0, 0)),
            pl.BlockSpec((1, D), lambda i: (0, 0)),
            pl.BlockSpec((D, D), lambda i: (0, 0)),
            pl.BlockSpec((1, D), lambda i: (0, 0)),
        ],
        out_specs=pl.BlockSpec((_B, D), lambda i: (0, 0)),
        out_shape=jax.ShapeDtypeStruct((_B, D), jnp.float32),
        scratch_shapes=[
            pltpu.VMEM((_B, _D), jnp.float32),
            pltpu.VMEM((nb, _B), jnp.float32),
            pltpu.VMEM((nb, _B), jnp.float32),
            pltpu.VMEM((nb * _B, _D), jnp.float32),
        ],
        compiler_params=pltpu.CompilerParams(
            dimension_semantics=("arbitrary",),
        ),
    )(seg3, feat_i, feat_u, W_user, bu, W_key, we)


# speedup vs baseline: 20.7928x; 1.7371x over previous
import functools
import jax
import jax.numpy as jnp
from jax.experimental import pallas as pl
from jax.experimental.pallas import tpu as pltpu

_B = 16
_D = 64


def _eye(dtype=jnp.float32):
    return (jax.lax.broadcasted_iota(jnp.int32, (_B, _B), 0)
            == jax.lax.broadcasted_iota(jnp.int32, (_B, _B), 1)).astype(dtype)


def _row_of(col):
    return jnp.sum(_eye() * col, axis=0, keepdims=True)


def _col_of(row):
    return jnp.sum(_eye() * row, axis=1, keepdims=True)


def _fused_body(nb, seg_ref, xT_ref, fuT_ref, wuT_ref, buT_ref, wkT_ref,
                weT_ref, out_ref, uT_scr, wkh_scr, m_parts, s_parts,
                acc_parts):
    i = pl.program_id(0)
    C = xT_ref.shape[1]

    @pl.when(i == 0)
    def _init():
        uT = jnp.dot(wuT_ref[...], fuT_ref[...],
                     preferred_element_type=jnp.float32) + buT_ref[...]
        uT_scr[...] = 0.5 * uT
        wkh_scr[...] = 0.5 * wkT_ref[...]

    xT = xT_ref[...]
    seg_row = seg_ref[0]
    onehot_b = seg_row == jax.lax.broadcasted_iota(jnp.int32, (_B, C), 0)
    onehot = onehot_b.astype(jnp.float32)

    keyT = jnp.dot(wkh_scr[...], xT, preferred_element_type=jnp.float32)
    qryT = jnp.dot(uT_scr[...], onehot, preferred_element_type=jnp.float32)
    thT = jnp.tanh(keyT + qryT)
    we_sum = jnp.sum(weT_ref[...], axis=1, keepdims=True)
    e_row = 0.5 * (jnp.dot(weT_ref[...], thT, preferred_element_type=jnp.float32)
                   + we_sum)

    em = jnp.where(onehot_b, jnp.broadcast_to(e_row, (_B, C)), -jnp.inf)
    cm_col = jnp.max(em, axis=1, keepdims=True)
    cm_safe = jnp.where(cm_col == -jnp.inf, 0.0, cm_col)
    cm_row = _row_of(cm_safe)
    row_cm = jnp.dot(cm_row, onehot, preferred_element_type=jnp.float32)
    p_row = jnp.exp(e_row - row_cm)
    whT = onehot * p_row
    s_col = jnp.sum(whT, axis=1, keepdims=True)
    acc_loc = jax.lax.dot_general(whT, xT, (((1,), (1,)), ((), ())),
                                  preferred_element_type=jnp.float32)

    m_parts[pl.ds(i, 1), :] = cm_row
    s_parts[pl.ds(i, 1), :] = _row_of(s_col)
    acc_parts[pl.ds(i * _B, _B), :] = acc_loc

    @pl.when(i == nb - 1)
    def _fin():
        mp = m_parts[...]
        sp = s_parts[...]
        m_row = jnp.max(jnp.where(sp > 0, mp, -jnp.inf), axis=0, keepdims=True)
        m_safe = jnp.where(m_row == -jnp.inf, 0.0, m_row)
        scp = jnp.exp(mp - m_safe)
        s_tot = jnp.sum(sp * scp, axis=0, keepdims=True)
        acc_tot = jnp.zeros((_B, _D), jnp.float32)
        for j in range(nb):
            sc_col = _col_of(scp[j:j + 1, :])
            acc_tot = acc_tot + sc_col * acc_parts[j * _B:(j + 1) * _B, :]
        s_ct = _col_of(s_tot)
        out_ref[...] = jnp.where(s_ct > 0, acc_tot / s_ct, 0.0)


def kernel(feat_i, feat_u, segment_ids, W_user, b_user, W_key, W_e):
    N, D = feat_i.shape
    C = 8192
    nb = N // C
    xT = feat_i.T
    fuT = feat_u.T
    wuT = W_user.T
    buT = b_user.reshape(D, 1)
    wkT = W_key.T
    weT = W_e.T
    seg3 = segment_ids.reshape(nb, 1, C)

    return pl.pallas_call(
        functools.partial(_fused_body, nb),
        grid=(nb,),
        in_specs=[
            pl.BlockSpec((1, 1, C), lambda i: (i, 0, 0)),
            pl.BlockSpec((D, C), lambda i: (0, i)),
            pl.BlockSpec((D, _B), lambda i: (0, 0)),
            pl.BlockSpec((D, D), lambda i: (0, 0)),
            pl.BlockSpec((D, 1), lambda i: (0, 0)),
            pl.BlockSpec((D, D), lambda i: (0, 0)),
            pl.BlockSpec((1, D), lambda i: (0, 0)),
        ],
        out_specs=pl.BlockSpec((_B, D), lambda i: (0, 0)),
        out_shape=jax.ShapeDtypeStruct((_B, D), jnp.float32),
        scratch_shapes=[
            pltpu.VMEM((_D, _B), jnp.float32),
            pltpu.VMEM((_D, _D), jnp.float32),
            pltpu.VMEM((nb, _B), jnp.float32),
            pltpu.VMEM((nb, _B), jnp.float32),
            pltpu.VMEM((nb * _B, _D), jnp.float32),
        ],
        compiler_params=pltpu.CompilerParams(
            dimension_semantics=("arbitrary",),
        ),
    )(seg3, xT, fuT, wuT, buT, wkT, weT)


# speedup vs baseline: 27.8919x; 1.3414x over previous
import functools
import jax
import jax.numpy as jnp
from jax.experimental import pallas as pl
from jax.experimental.pallas import tpu as pltpu

_B = 16
_D = 64


def _eye(dtype=jnp.float32):
    return (jax.lax.broadcasted_iota(jnp.int32, (_B, _B), 0)
            == jax.lax.broadcasted_iota(jnp.int32, (_B, _B), 1)).astype(dtype)


def _row_of(col):
    return jnp.sum(_eye() * col, axis=0, keepdims=True)


def _col_of(row):
    return jnp.sum(_eye() * row, axis=1, keepdims=True)


def _fused_body(nb, seg_ref, xT_ref, fu_ref, wu_ref, bu_ref, wk_ref,
                we_ref, out_ref, uT_scr, wkh_scr, weT_scr, m_parts, s_parts,
                acc_parts):
    i = pl.program_id(0)
    C = xT_ref.shape[1]

    @pl.when(i == 0)
    def _init():
        eye16 = _eye()
        eye64 = (jax.lax.broadcasted_iota(jnp.int32, (_D, _D), 0)
                 == jax.lax.broadcasted_iota(jnp.int32, (_D, _D), 1)
                 ).astype(jnp.float32)
        u = jnp.dot(fu_ref[...], wu_ref[...],
                    preferred_element_type=jnp.float32) + bu_ref[...]
        uT_scr[...] = 0.5 * jax.lax.dot_general(
            u, eye16, (((0,), (0,)), ((), ())),
            preferred_element_type=jnp.float32)
        wkh_scr[...] = 0.5 * jax.lax.dot_general(
            wk_ref[...], eye64, (((0,), (0,)), ((), ())),
            preferred_element_type=jnp.float32)
        weT_scr[...] = jnp.sum(eye64 * we_ref[...], axis=0, keepdims=True)

    xT = xT_ref[...]
    seg_row = seg_ref[0]
    onehot_b = seg_row == jax.lax.broadcasted_iota(jnp.int32, (_B, C), 0)
    onehot = onehot_b.astype(jnp.float32)

    keyT = jnp.dot(wkh_scr[...], xT, preferred_element_type=jnp.float32)
    qryT = jnp.dot(uT_scr[...], onehot, preferred_element_type=jnp.float32)
    thT = jnp.tanh(keyT + qryT)
    we_sum = jnp.sum(weT_scr[...], axis=1, keepdims=True)
    e_row = 0.5 * (jnp.dot(weT_scr[...], thT, preferred_element_type=jnp.float32)
                   + we_sum)

    em = jnp.where(onehot_b, jnp.broadcast_to(e_row, (_B, C)), -jnp.inf)
    cm_col = jnp.max(em, axis=1, keepdims=True)
    cm_safe = jnp.where(cm_col == -jnp.inf, 0.0, cm_col)
    cm_row = _row_of(cm_safe)
    row_cm = jnp.dot(cm_row, onehot, preferred_element_type=jnp.float32)
    p_row = jnp.exp(e_row - row_cm)
    whT = onehot * p_row
    s_col = jnp.sum(whT, axis=1, keepdims=True)
    acc_loc = jax.lax.dot_general(whT, xT, (((1,), (1,)), ((), ())),
                                  preferred_element_type=jnp.float32)

    m_parts[pl.ds(i, 1), :] = cm_row
    s_parts[pl.ds(i, 1), :] = _row_of(s_col)
    acc_parts[pl.ds(i * _B, _B), :] = acc_loc

    @pl.when(i == nb - 1)
    def _fin():
        mp = m_parts[...]
        sp = s_parts[...]
        m_row = jnp.max(jnp.where(sp > 0, mp, -jnp.inf), axis=0, keepdims=True)
        m_safe = jnp.where(m_row == -jnp.inf, 0.0, m_row)
        scp = jnp.exp(mp - m_safe)
        s_tot = jnp.sum(sp * scp, axis=0, keepdims=True)
        acc_tot = jnp.zeros((_B, _D), jnp.float32)
        for j in range(nb):
            sc_col = _col_of(scp[j:j + 1, :])
            acc_tot = acc_tot + sc_col * acc_parts[j * _B:(j + 1) * _B, :]
        s_ct = _col_of(s_tot)
        out_ref[...] = jnp.where(s_ct > 0, acc_tot / s_ct, 0.0)


def kernel(feat_i, feat_u, segment_ids, W_user, b_user, W_key, W_e):
    N, D = feat_i.shape
    C = 8192
    nb = N // C
    xT = feat_i.T
    bu = b_user.reshape(1, D)
    seg3 = segment_ids.reshape(nb, 1, C)

    return pl.pallas_call(
        functools.partial(_fused_body, nb),
        grid=(nb,),
        in_specs=[
            pl.BlockSpec((1, 1, C), lambda i: (i, 0, 0)),
            pl.BlockSpec((D, C), lambda i: (0, i)),
            pl.BlockSpec((_B, D), lambda i: (0, 0)),
            pl.BlockSpec((D, D), lambda i: (0, 0)),
            pl.BlockSpec((1, D), lambda i: (0, 0)),
            pl.BlockSpec((D, D), lambda i: (0, 0)),
            pl.BlockSpec((D, 1), lambda i: (0, 0)),
        ],
        out_specs=pl.BlockSpec((_B, D), lambda i: (0, 0)),
        out_shape=jax.ShapeDtypeStruct((_B, D), jnp.float32),
        scratch_shapes=[
            pltpu.VMEM((_D, _B), jnp.float32),
            pltpu.VMEM((_D, _D), jnp.float32),
            pltpu.VMEM((1, _D), jnp.float32),
            pltpu.VMEM((nb, _B), jnp.float32),
            pltpu.VMEM((nb, _B), jnp.float32),
            pltpu.VMEM((nb * _B, _D), jnp.float32),
        ],
        compiler_params=pltpu.CompilerParams(
            dimension_semantics=("arbitrary",),
        ),
    )(seg3, xT, feat_u, W_user, bu, W_key, W_e)


# speedup vs baseline: 27.9595x; 1.0024x over previous
import functools
import jax
import jax.numpy as jnp
from jax.experimental import pallas as pl
from jax.experimental.pallas import tpu as pltpu

_B = 16
_D = 64


def _eye(dtype=jnp.float32):
    return (jax.lax.broadcasted_iota(jnp.int32, (_B, _B), 0)
            == jax.lax.broadcasted_iota(jnp.int32, (_B, _B), 1)).astype(dtype)


def _row_of(col):
    return jnp.sum(_eye() * col, axis=0, keepdims=True)


def _col_of(row):
    return jnp.sum(_eye() * row, axis=1, keepdims=True)


def _fused_body(nb, seg_ref, xT_ref, fu_ref, wu_ref, bu_ref, wk_ref,
                we_ref, out_ref, uT_scr, wkh_scr, weT_scr, m_parts, s_parts,
                acc_parts):
    i = pl.program_id(0)
    C = xT_ref.shape[1]

    @pl.when(i == 0)
    def _init():
        eye16 = _eye()
        eye64 = (jax.lax.broadcasted_iota(jnp.int32, (_D, _D), 0)
                 == jax.lax.broadcasted_iota(jnp.int32, (_D, _D), 1)
                 ).astype(jnp.float32)
        u = jnp.dot(fu_ref[...], wu_ref[...],
                    preferred_element_type=jnp.float32) + bu_ref[...]
        uT_scr[...] = 0.5 * jax.lax.dot_general(
            u, eye16, (((0,), (0,)), ((), ())),
            preferred_element_type=jnp.float32)
        wkh_scr[...] = 0.5 * jax.lax.dot_general(
            wk_ref[...], eye64, (((0,), (0,)), ((), ())),
            preferred_element_type=jnp.float32)
        weT_scr[...] = jnp.sum(eye64 * we_ref[...], axis=0, keepdims=True)

    xT = xT_ref[...]
    seg_row = seg_ref[0]
    onehot_b = seg_row == jax.lax.broadcasted_iota(jnp.int32, (_B, C), 0)
    onehot = onehot_b.astype(jnp.float32)

    keyT = jnp.dot(wkh_scr[...], xT, preferred_element_type=jnp.float32,
                   precision=jax.lax.Precision.DEFAULT)
    qryT = jnp.dot(uT_scr[...], onehot, preferred_element_type=jnp.float32,
                   precision=jax.lax.Precision.DEFAULT)
    thT = jnp.tanh(keyT + qryT)
    we_sum = jnp.sum(weT_scr[...], axis=1, keepdims=True)
    e_row = 0.5 * (jnp.dot(weT_scr[...], thT, preferred_element_type=jnp.float32,
                           precision=jax.lax.Precision.DEFAULT)
                   + we_sum)

    em = jnp.where(onehot_b, jnp.broadcast_to(e_row, (_B, C)), -jnp.inf)
    cm_col = jnp.max(em, axis=1, keepdims=True)
    cm_safe = jnp.where(cm_col == -jnp.inf, 0.0, cm_col)
    cm_row = _row_of(cm_safe)
    row_cm = jnp.dot(cm_row, onehot, preferred_element_type=jnp.float32)
    p_row = jnp.exp(e_row - row_cm)
    whT = onehot * p_row
    s_col = jnp.sum(whT, axis=1, keepdims=True)
    acc_loc = jax.lax.dot_general(whT, xT, (((1,), (1,)), ((), ())),
                                  preferred_element_type=jnp.float32)

    m_parts[pl.ds(i, 1), :] = cm_row
    s_parts[pl.ds(i, 1), :] = _row_of(s_col)
    acc_parts[pl.ds(i * _B, _B), :] = acc_loc

    @pl.when(i == nb - 1)
    def _fin():
        mp = m_parts[...]
        sp = s_parts[...]
        m_row = jnp.max(jnp.where(sp > 0, mp, -jnp.inf), axis=0, keepdims=True)
        m_safe = jnp.where(m_row == -jnp.inf, 0.0, m_row)
        scp = jnp.exp(mp - m_safe)
        s_tot = jnp.sum(sp * scp, axis=0, keepdims=True)
        acc_tot = jnp.zeros((_B, _D), jnp.float32)
        for j in range(nb):
            sc_col = _col_of(scp[j:j + 1, :])
            acc_tot = acc_tot + sc_col * acc_parts[j * _B:(j + 1) * _B, :]
        s_ct = _col_of(s_tot)
        out_ref[...] = jnp.where(s_ct > 0, acc_tot / s_ct, 0.0)


def kernel(feat_i, feat_u, segment_ids, W_user, b_user, W_key, W_e):
    N, D = feat_i.shape
    C = 8192
    nb = N // C
    xT = feat_i.T
    bu = b_user.reshape(1, D)
    seg3 = segment_ids.reshape(nb, 1, C)

    return pl.pallas_call(
        functools.partial(_fused_body, nb),
        grid=(nb,),
        in_specs=[
            pl.BlockSpec((1, 1, C), lambda i: (i, 0, 0)),
            pl.BlockSpec((D, C), lambda i: (0, i)),
            pl.BlockSpec((_B, D), lambda i: (0, 0)),
            pl.BlockSpec((D, D), lambda i: (0, 0)),
            pl.BlockSpec((1, D), lambda i: (0, 0)),
            pl.BlockSpec((D, D), lambda i: (0, 0)),
            pl.BlockSpec((D, 1), lambda i: (0, 0)),
        ],
        out_specs=pl.BlockSpec((_B, D), lambda i: (0, 0)),
        out_shape=jax.ShapeDtypeStruct((_B, D), jnp.float32),
        scratch_shapes=[
            pltpu.VMEM((_D, _B), jnp.float32),
            pltpu.VMEM((_D, _D), jnp.float32),
            pltpu.VMEM((1, _D), jnp.float32),
            pltpu.VMEM((nb, _B), jnp.float32),
            pltpu.VMEM((nb, _B), jnp.float32),
            pltpu.VMEM((nb * _B, _D), jnp.float32),
        ],
        compiler_params=pltpu.CompilerParams(
            dimension_semantics=("arbitrary",),
        ),
    )(seg3, xT, feat_u, W_user, bu, W_key, W_e)


# speedup vs baseline: 28.3532x; 1.0141x over previous
import functools
import jax
import jax.numpy as jnp
from jax.experimental import pallas as pl
from jax.experimental.pallas import tpu as pltpu

_B = 16
_D = 64


def _eye(dtype=jnp.float32):
    return (jax.lax.broadcasted_iota(jnp.int32, (_B, _B), 0)
            == jax.lax.broadcasted_iota(jnp.int32, (_B, _B), 1)).astype(dtype)


def _row_of(col):
    return jnp.sum(_eye() * col, axis=0, keepdims=True)


def _col_of(row):
    return jnp.sum(_eye() * row, axis=1, keepdims=True)


def _fused_body(nb, seg_ref, xT_ref, fu_ref, wu_ref, bu_ref, wk_ref,
                we_ref, out_ref, uT_scr, wkh_scr, weT_scr, m_parts, s_parts,
                acc_parts):
    i = pl.program_id(0)
    C = xT_ref.shape[1]

    @pl.when(i == 0)
    def _init():
        eye16 = _eye()
        eye64 = (jax.lax.broadcasted_iota(jnp.int32, (_D, _D), 0)
                 == jax.lax.broadcasted_iota(jnp.int32, (_D, _D), 1)
                 ).astype(jnp.float32)
        u = jnp.dot(fu_ref[...], wu_ref[...],
                    preferred_element_type=jnp.float32) + bu_ref[...]
        uT_scr[...] = 0.5 * jax.lax.dot_general(
            u, eye16, (((0,), (0,)), ((), ())),
            preferred_element_type=jnp.float32)
        wkh_scr[...] = 0.5 * jax.lax.dot_general(
            wk_ref[...], eye64, (((0,), (0,)), ((), ())),
            preferred_element_type=jnp.float32)
        weT_scr[...] = jnp.sum(eye64 * we_ref[...], axis=0, keepdims=True)

    xT = xT_ref[...]
    seg_row = seg_ref[0]
    onehot_b = seg_row == jax.lax.broadcasted_iota(jnp.int32, (_B, C), 0)
    onehot = onehot_b.astype(jnp.float32)

    keyT = jnp.dot(wkh_scr[...], xT, preferred_element_type=jnp.float32,
                   precision=jax.lax.Precision.DEFAULT)
    qryT = jnp.dot(uT_scr[...], onehot, preferred_element_type=jnp.float32,
                   precision=jax.lax.Precision.DEFAULT)
    thT = jnp.tanh(keyT + qryT)
    we_sum = jnp.sum(weT_scr[...], axis=1, keepdims=True)
    e_row = 0.5 * (jnp.dot(weT_scr[...], thT, preferred_element_type=jnp.float32,
                           precision=jax.lax.Precision.DEFAULT)
                   + we_sum)

    em = jnp.where(onehot_b, jnp.broadcast_to(e_row, (_B, C)), -jnp.inf)
    cm_col = jnp.max(em, axis=1, keepdims=True)
    cm_safe = jnp.where(cm_col == -jnp.inf, 0.0, cm_col)
    cm_row = _row_of(cm_safe)
    row_cm = jnp.dot(cm_row, onehot, preferred_element_type=jnp.float32)
    p_row = jnp.exp(e_row - row_cm)
    whT = onehot * p_row
    s_col = jnp.sum(whT, axis=1, keepdims=True)
    acc_loc = jax.lax.dot_general(whT, xT, (((1,), (1,)), ((), ())),
                                  preferred_element_type=jnp.float32)

    m_parts[pl.ds(i, 1), :] = cm_row
    s_parts[pl.ds(i, 1), :] = _row_of(s_col)
    acc_parts[pl.ds(i * _B, _B), :] = acc_loc

    @pl.when(i == nb - 1)
    def _fin():
        mp = m_parts[...]
        sp = s_parts[...]
        m_row = jnp.max(jnp.where(sp > 0, mp, -jnp.inf), axis=0, keepdims=True)
        m_safe = jnp.where(m_row == -jnp.inf, 0.0, m_row)
        scp = jnp.exp(mp - m_safe)
        s_tot = jnp.sum(sp * scp, axis=0, keepdims=True)
        acc_tot = jnp.zeros((_B, _D), jnp.float32)
        for j in range(nb):
            sc_col = _col_of(scp[j:j + 1, :])
            acc_tot = acc_tot + sc_col * acc_parts[j * _B:(j + 1) * _B, :]
        s_ct = _col_of(s_tot)
        out_ref[...] = jnp.where(s_ct > 0, acc_tot / s_ct, 0.0)


def kernel(feat_i, feat_u, segment_ids, W_user, b_user, W_key, W_e):
    N, D = feat_i.shape
    C = 16384
    nb = N // C
    xT = feat_i.T
    bu = b_user.reshape(1, D)
    seg3 = segment_ids.reshape(nb, 1, C)

    return pl.pallas_call(
        functools.partial(_fused_body, nb),
        grid=(nb,),
        in_specs=[
            pl.BlockSpec((1, 1, C), lambda i: (i, 0, 0)),
            pl.BlockSpec((D, C), lambda i: (0, i)),
            pl.BlockSpec((_B, D), lambda i: (0, 0)),
            pl.BlockSpec((D, D), lambda i: (0, 0)),
            pl.BlockSpec((1, D), lambda i: (0, 0)),
            pl.BlockSpec((D, D), lambda i: (0, 0)),
            pl.BlockSpec((D, 1), lambda i: (0, 0)),
        ],
        out_specs=pl.BlockSpec((_B, D), lambda i: (0, 0)),
        out_shape=jax.ShapeDtypeStruct((_B, D), jnp.float32),
        scratch_shapes=[
            pltpu.VMEM((_D, _B), jnp.float32),
            pltpu.VMEM((_D, _D), jnp.float32),
            pltpu.VMEM((1, _D), jnp.float32),
            pltpu.VMEM((nb, _B), jnp.float32),
            pltpu.VMEM((nb, _B), jnp.float32),
            pltpu.VMEM((nb * _B, _D), jnp.float32),
        ],
        compiler_params=pltpu.CompilerParams(
            dimension_semantics=("arbitrary",),
        ),
    )(seg3, xT, feat_u, W_user, bu, W_key, W_e)


# speedup vs baseline: 37.7636x; 1.3319x over previous
import functools
import jax
import jax.numpy as jnp
from jax.experimental import pallas as pl
from jax.experimental.pallas import tpu as pltpu

_B = 16
_D = 64


def _eye(n, dtype=jnp.float32):
    return (jax.lax.broadcasted_iota(jnp.int32, (n, n), 0)
            == jax.lax.broadcasted_iota(jnp.int32, (n, n), 1)).astype(dtype)


def _fused_body(nb, seg_ref, xT_ref, fu_ref, wu_ref, bu_ref, wk_ref,
                we_ref, out_ref, uT_scr, wkh_scr, weT_scr, m_scr,
                s_scr, acc_scr):
    i = pl.program_id(0)
    C = xT_ref.shape[1]

    @pl.when(i == 0)
    def _init():
        eye16 = _eye(_B)
        eye64 = _eye(_D)
        u = jnp.dot(fu_ref[...], wu_ref[...],
                    preferred_element_type=jnp.float32) + bu_ref[...]
        uT_scr[...] = 0.5 * jax.lax.dot_general(
            u, eye16, (((0,), (0,)), ((), ())),
            preferred_element_type=jnp.float32)
        wkh_scr[...] = 0.5 * jax.lax.dot_general(
            wk_ref[...], eye64, (((0,), (0,)), ((), ())),
            preferred_element_type=jnp.float32)
        weT = jnp.sum(eye64 * we_ref[...], axis=0, keepdims=True)
        weT_scr[...] = weT
        we_sum = jnp.sum(weT, axis=1, keepdims=True)
        m_bound = jnp.sum(jnp.maximum(weT, 0.0), axis=1, keepdims=True)
        m_scr[0:1, 0:1] = m_bound
        m_scr[1:2, 0:1] = we_sum
        s_scr[...] = jnp.zeros(s_scr.shape, s_scr.dtype)
        acc_scr[...] = jnp.zeros(acc_scr.shape, acc_scr.dtype)

    xT = xT_ref[...]
    seg_row = seg_ref[0]
    onehot_b = seg_row == jax.lax.broadcasted_iota(jnp.int32, (_B, C), 0)
    onehot = onehot_b.astype(jnp.float32)

    keyT = jnp.dot(wkh_scr[...], xT, preferred_element_type=jnp.float32)
    qryT = jnp.dot(uT_scr[...], onehot, preferred_element_type=jnp.float32)
    thT = jnp.tanh(keyT + qryT)
    e2_row = (jnp.dot(weT_scr[...], thT, preferred_element_type=jnp.float32)
              + m_scr[1:2, 0:1])
    p_row = jnp.exp(0.5 * e2_row - m_scr[0:1, 0:1])
    whT = onehot * p_row
    s_new = s_scr[...] + jnp.sum(whT, axis=1, keepdims=True)
    acc_new = acc_scr[...] + jax.lax.dot_general(
        whT, xT, (((1,), (1,)), ((), ())),
        preferred_element_type=jnp.float32)
    s_scr[...] = s_new
    acc_scr[...] = acc_new

    @pl.when(i == nb - 1)
    def _fin():
        out_ref[...] = jnp.where(s_new > 0, acc_new / s_new, 0.0)


def kernel(feat_i, feat_u, segment_ids, W_user, b_user, W_key, W_e):
    N, D = feat_i.shape
    C = 16384
    nb = N // C
    xT = feat_i.T
    bu = b_user.reshape(1, D)
    seg3 = segment_ids.reshape(nb, 1, C)

    return pl.pallas_call(
        functools.partial(_fused_body, nb),
        grid=(nb,),
        in_specs=[
            pl.BlockSpec((1, 1, C), lambda i: (i, 0, 0)),
            pl.BlockSpec((D, C), lambda i: (0, i)),
            pl.BlockSpec((_B, D), lambda i: (0, 0)),
            pl.BlockSpec((D, D), lambda i: (0, 0)),
            pl.BlockSpec((1, D), lambda i: (0, 0)),
            pl.BlockSpec((D, D), lambda i: (0, 0)),
            pl.BlockSpec((D, 1), lambda i: (0, 0)),
        ],
        out_specs=pl.BlockSpec((_B, D), lambda i: (0, 0)),
        out_shape=jax.ShapeDtypeStruct((_B, D), jnp.float32),
        scratch_shapes=[
            pltpu.VMEM((_D, _B), jnp.float32),
            pltpu.VMEM((_D, _D), jnp.float32),
            pltpu.VMEM((1, _D), jnp.float32),
            pltpu.VMEM((2, 1), jnp.float32),
            pltpu.VMEM((_B, 1), jnp.float32),
            pltpu.VMEM((_B, _D), jnp.float32),
        ],
        compiler_params=pltpu.CompilerParams(
            dimension_semantics=("arbitrary",),
        ),
    )(seg3, xT, feat_u, W_user, bu, W_key, W_e)
